# Initial kernel scaffold; baseline (speedup 1.0000x reference)
import jax
import jax.numpy as jnp
from jax.experimental import pallas as pl


def kernel(x, edge_index_l0, edge_index_l1, W0, b0, W1, b1):
    raise NotImplementedError("write your pallas kernel here")


# speedup vs baseline: 7.0084x; 7.0084x over previous
import functools

import jax
import jax.numpy as jnp
from jax import lax
from jax.experimental import pallas as pl
from jax.experimental.pallas import tpu as pltpu
from jax.experimental.pallas import tpu_sc as plsc

N = 10000
NPAD = 10240
E = 320000
NC, NS = 2, 16
ET = E // NS
CHUNK = 128
NCHUNK = -(-ET // CHUNK)
ETP = NCHUNK * CHUNK
HCHUNK = 125
NHCHUNK = ET // HCHUNK
ROWS_T = NPAD // NS

_mesh = plsc.VectorSubcoreMesh(
    core_axis_name="c", subcore_axis_name="s", num_cores=NC, num_subcores=NS)


def _hist_body(idx_hbm, out_hbm, idx_v, ones_v, zero_v, hist_a, hist_b):
    cid = lax.axis_index("c")
    sid = lax.axis_index("s")

    @pl.loop(0, 8)
    def _(i):
        ones_v[pl.ds(i * 16, 16)] = jnp.ones((16,), jnp.float32)

    @pl.loop(0, ROWS_T // 16)
    def _(i):
        zero_v[pl.ds(i * 16, 16)] = jnp.zeros((16,), jnp.float32)

    pltpu.sync_copy(zero_v, hist_a.at[pl.ds(sid * ROWS_T, ROWS_T)])
    pltpu.sync_copy(zero_v, hist_b.at[pl.ds(sid * ROWS_T, ROWS_T)])
    plsc.subcore_barrier()

    for k, hist in ((0, hist_a), (1, hist_b)):
        m = (2 * cid + k) * NS + sid
        pltpu.sync_copy(idx_hbm.at[m], idx_v)

        @pl.loop(0, NHCHUNK)
        def _(j):
            pltpu.sync_copy(ones_v.at[pl.ds(0, HCHUNK)],
                            hist.at[idx_v.at[j]], add=True)

    plsc.subcore_barrier()
    pltpu.sync_copy(hist_a.at[pl.ds(sid * ROWS_T, ROWS_T)],
                    out_hbm.at[pl.ds((2 * cid) * NPAD + sid * ROWS_T, ROWS_T)])
    pltpu.sync_copy(hist_b.at[pl.ds(sid * ROWS_T, ROWS_T)],
                    out_hbm.at[pl.ds((2 * cid + 1) * NPAD + sid * ROWS_T, ROWS_T)])


_hist_call = functools.partial(
    pl.kernel,
    out_type=jax.ShapeDtypeStruct((4 * NPAD,), jnp.float32),
    mesh=_mesh,
    scratch_types=[
        pltpu.VMEM((NHCHUNK, HCHUNK), jnp.int32),
        pltpu.VMEM((128,), jnp.float32),
        pltpu.VMEM((ROWS_T,), jnp.float32),
        pltpu.VMEM_SHARED((NPAD,), jnp.float32),
        pltpu.VMEM_SHARED((NPAD,), jnp.float32),
    ],
)(_hist_body)


def _make_agg(DH):
    def body(feat_hbm, src_hbm, dst_hbm, out_hbm, src_v, dst_v, buf, acc_sh):
        cid = lax.axis_index("c")
        sid = lax.axis_index("s")

        @pl.loop(0, 128)
        def _(r):
            for c in range(DH // 16):
                buf[r, pl.ds(c * 16, 16)] = jnp.zeros((16,), jnp.float32)

        for t in range(ROWS_T // 128):
            pltpu.sync_copy(buf, acc_sh.at[pl.ds(sid * ROWS_T + t * 128, 128)])

        pltpu.sync_copy(src_hbm.at[sid], src_v)
        pltpu.sync_copy(dst_hbm.at[sid], dst_v)

        off = cid * NPAD

        @pl.loop(0, NCHUNK)
        def _(j):
            for c in range(CHUNK // 16):
                sl = pl.ds(c * 16, 16)
                src_v[j, sl] = src_v[j, sl] + off

        plsc.subcore_barrier()

        @pl.loop(0, NCHUNK)
        def _(j):
            pltpu.sync_copy(feat_hbm.at[src_v.at[j]], buf)
            pltpu.sync_copy(buf, acc_sh.at[dst_v.at[j]], add=True)

        plsc.subcore_barrier()
        for t in range(ROWS_T // 128):
            base = sid * ROWS_T + t * 128
            pltpu.sync_copy(acc_sh.at[pl.ds(base, 128)],
                            out_hbm.at[pl.ds(cid * NPAD + base, 128)])

    return functools.partial(
        pl.kernel,
        out_type=jax.ShapeDtypeStruct((NC * NPAD, DH), jnp.float32),
        mesh=_mesh,
        scratch_types=[
            pltpu.VMEM((NCHUNK, CHUNK), jnp.int32),
            pltpu.VMEM((NCHUNK, CHUNK), jnp.int32),
            pltpu.VMEM((CHUNK, DH), jnp.float32),
            pltpu.VMEM_SHARED((NPAD, DH), jnp.float32),
        ],
        compiler_params=pltpu.CompilerParams(use_tc_tiling_on_sc=False),
    )(body)


_agg_hid = _make_agg(64)
_agg_out = _make_agg(32)


def _feat0_body(x_ref, deg_ref, w_ref, o_ref):
    norm = lax.rsqrt(jnp.maximum(deg_ref[...], 1.0))
    o_ref[0] = jnp.dot(x_ref[...] * norm, w_ref[0],
                       preferred_element_type=jnp.float32)


def _mid_body(p_ref, degi_ref, dego_ref, b0_ref, w1_ref, o_ref):
    agg = jnp.concatenate([p_ref[0], p_ref[1]], axis=1)
    ni = lax.rsqrt(jnp.maximum(degi_ref[...], 1.0))
    h = jnp.maximum(agg * ni + b0_ref[...], 0.0)
    ns = lax.rsqrt(jnp.maximum(dego_ref[...], 1.0))
    o_ref[0] = jnp.dot(h * ns, w1_ref[0],
                       preferred_element_type=jnp.float32)


def _fin_body(p_ref, degi_ref, b1_ref, o_ref):
    agg = jnp.concatenate([p_ref[0], p_ref[1]], axis=1)
    ni = lax.rsqrt(jnp.maximum(degi_ref[...], 1.0))
    o_ref[...] = agg * ni + b1_ref[...]


_BLK = 1024
_GRID = NPAD // _BLK


def _feat0(x_pad, deg, W0):
    return pl.pallas_call(
        _feat0_body,
        grid=(2, _GRID),
        in_specs=[
            pl.BlockSpec((_BLK, 128), lambda j, i: (i, 0)),
            pl.BlockSpec((_BLK, 1), lambda j, i: (i, 0)),
            pl.BlockSpec((1, 128, 64), lambda j, i: (j, 0, 0)),
        ],
        out_specs=pl.BlockSpec((1, _BLK, 64), lambda j, i: (j, i, 0)),
        out_shape=jax.ShapeDtypeStruct((NC, NPAD, 64), jnp.float32),
    )(x_pad, deg, W0)


def _mid(p, degi, dego, b0, W1):
    return pl.pallas_call(
        _mid_body,
        grid=(2, _GRID),
        in_specs=[
            pl.BlockSpec((NC, _BLK, 64), lambda j, i: (0, i, 0)),
            pl.BlockSpec((_BLK, 1), lambda j, i: (i, 0)),
            pl.BlockSpec((_BLK, 1), lambda j, i: (i, 0)),
            pl.BlockSpec((1, 128), lambda j, i: (0, 0)),
            pl.BlockSpec((1, 128, 32), lambda j, i: (j, 0, 0)),
        ],
        out_specs=pl.BlockSpec((1, _BLK, 32), lambda j, i: (j, i, 0)),
        out_shape=jax.ShapeDtypeStruct((NC, NPAD, 32), jnp.float32),
    )(p, degi, dego, b0, W1)


def _fin(p, degi, b1):
    return pl.pallas_call(
        _fin_body,
        grid=(_GRID,),
        in_specs=[
            pl.BlockSpec((NC, _BLK, 32), lambda i: (0, i, 0)),
            pl.BlockSpec((_BLK, 1), lambda i: (i, 0)),
            pl.BlockSpec((1, 64), lambda i: (0, 0)),
        ],
        out_specs=pl.BlockSpec((_BLK, 64), lambda i: (i, 0)),
        out_shape=jax.ShapeDtypeStruct((NPAD, 64), jnp.float32),
    )(p, degi, b1)


def _edge_split(idx, pad_value):
    t = idx.reshape(NS, ET)
    t = jnp.pad(t, ((0, 0), (0, ETP - ET)), constant_values=pad_value)
    return t.reshape(NS, NCHUNK, CHUNK)


def kernel(x, edge_index_l0, edge_index_l1, W0, b0, W1, b1):
    src0 = edge_index_l0[0].astype(jnp.int32)
    dst0 = edge_index_l0[1].astype(jnp.int32)
    src1 = edge_index_l1[0].astype(jnp.int32)
    dst1 = edge_index_l1[1].astype(jnp.int32)

    hist_in = jnp.stack([src0, dst0, src1, dst1]).reshape(
        4 * NS, NHCHUNK, HCHUNK)
    hist = _hist_call(hist_in).reshape(4, NPAD)
    deg_src0 = hist[0].reshape(NPAD, 1)
    deg_dst0 = hist[1].reshape(NPAD, 1)
    deg_src1 = hist[2].reshape(NPAD, 1)
    deg_dst1 = hist[3].reshape(NPAD, 1)

    x_pad = jnp.pad(x, ((0, NPAD - N), (0, 0)))
    W0h = jnp.stack([W0[:, :64], W0[:, 64:]])
    W1h = jnp.stack([W1[:, :32], W1[:, 32:]])
    feat0 = _feat0(x_pad, deg_src0, W0h).reshape(NC * NPAD, 64)

    p0 = _agg_hid(feat0, _edge_split(src0, 0),
                  _edge_split(dst0, N)).reshape(NC, NPAD, 64)
    feat1 = _mid(p0, deg_dst0, deg_src1,
                 b0.reshape(1, 128), W1h).reshape(NC * NPAD, 32)

    p1 = _agg_out(feat1, _edge_split(src1, 0),
                  _edge_split(dst1, N)).reshape(NC, NPAD, 32)
    out = _fin(p1, deg_dst1, b1.reshape(1, 64))

    total_flops = float(
        (2 * 128 * E + 2 * 128 * 128 * N + N * 128) / 1e12
        + (2 * 128 * E + 2 * 128 * 64 * N + N * 128) / 1e12)
    return out[:N], total_flops


# speedup vs baseline: 9.7187x; 1.3867x over previous
import functools

import jax
import jax.numpy as jnp
from jax import lax
from jax.experimental import pallas as pl
from jax.experimental.pallas import tpu as pltpu
from jax.experimental.pallas import tpu_sc as plsc

N = 10000
NPAD = 10240
E = 320000
NC, NS = 2, 16
ET = E // NS
CHUNK = 128
NCHUNK = -(-ET // CHUNK)
ETP = NCHUNK * CHUNK
HCHUNK = 125
NHCHUNK = ET // HCHUNK
ROWS_T = NPAD // NS

_mesh = plsc.VectorSubcoreMesh(
    core_axis_name="c", subcore_axis_name="s", num_cores=NC, num_subcores=NS)


def _hist_body(idx_hbm, out_hbm, idx_v, ones_v, zero_v, hist_a, hist_b):
    cid = lax.axis_index("c")
    sid = lax.axis_index("s")

    @pl.loop(0, 8)
    def _(i):
        ones_v[pl.ds(i * 16, 16)] = jnp.ones((16,), jnp.float32)

    @pl.loop(0, ROWS_T // 16)
    def _(i):
        zero_v[pl.ds(i * 16, 16)] = jnp.zeros((16,), jnp.float32)

    pltpu.sync_copy(zero_v, hist_a.at[pl.ds(sid * ROWS_T, ROWS_T)])
    pltpu.sync_copy(zero_v, hist_b.at[pl.ds(sid * ROWS_T, ROWS_T)])
    plsc.subcore_barrier()

    for k, hist in ((0, hist_a), (1, hist_b)):
        m = (2 * cid + k) * NS + sid
        pltpu.sync_copy(idx_hbm.at[m], idx_v)

        @pl.loop(0, NHCHUNK)
        def _(j):
            pltpu.sync_copy(ones_v.at[pl.ds(0, HCHUNK)],
                            hist.at[idx_v.at[j]], add=True)

    plsc.subcore_barrier()
    pltpu.sync_copy(hist_a.at[pl.ds(sid * ROWS_T, ROWS_T)],
                    out_hbm.at[pl.ds((2 * cid) * NPAD + sid * ROWS_T, ROWS_T)])
    pltpu.sync_copy(hist_b.at[pl.ds(sid * ROWS_T, ROWS_T)],
                    out_hbm.at[pl.ds((2 * cid + 1) * NPAD + sid * ROWS_T, ROWS_T)])


_hist_call = functools.partial(
    pl.kernel,
    out_type=jax.ShapeDtypeStruct((4 * NPAD,), jnp.float32),
    mesh=_mesh,
    scratch_types=[
        pltpu.VMEM((NHCHUNK, HCHUNK), jnp.int32),
        pltpu.VMEM((128,), jnp.float32),
        pltpu.VMEM((ROWS_T,), jnp.float32),
        pltpu.VMEM_SHARED((NPAD,), jnp.float32),
        pltpu.VMEM_SHARED((NPAD,), jnp.float32),
    ],
)(_hist_body)


def _make_agg(DH):
    def body(feat_hbm, src_hbm, dst_hbm, out_hbm, src_v, dst_v, buf0, buf1,
             acc_sh, sem0, sem1):
        cid = lax.axis_index("c")
        sid = lax.axis_index("s")

        @pl.loop(0, 128)
        def _(r):
            for c in range(DH // 16):
                buf0[r, pl.ds(c * 16, 16)] = jnp.zeros((16,), jnp.float32)

        for t in range(ROWS_T // 128):
            pltpu.sync_copy(buf0, acc_sh.at[pl.ds(sid * ROWS_T + t * 128, 128)])

        pltpu.sync_copy(src_hbm.at[sid], src_v)
        pltpu.sync_copy(dst_hbm.at[sid], dst_v)

        off = cid * NPAD

        @pl.loop(0, NCHUNK)
        def _(j):
            for c in range(CHUNK // 16):
                sl = pl.ds(c * 16, 16)
                src_v[j, sl] = src_v[j, sl] + off

        plsc.subcore_barrier()

        def gstart(j, buf, sem):
            pltpu.async_copy(feat_hbm.at[src_v.at[j]], buf, sem)

        def gwait(j, buf, sem):
            pltpu.make_async_copy(feat_hbm.at[src_v.at[j]], buf, sem).wait()

        def scat(j, buf):
            pltpu.sync_copy(buf, acc_sh.at[dst_v.at[j]], add=True)

        gstart(0, buf0, sem0)
        gstart(1, buf1, sem1)

        @pl.loop(0, (NCHUNK - 3) // 2)
        def _(i):
            j = 2 * i
            gwait(j, buf0, sem0)
            scat(j, buf0)
            gstart(j + 2, buf0, sem0)
            gwait(j + 1, buf1, sem1)
            scat(j + 1, buf1)
            gstart(j + 3, buf1, sem1)

        gwait(NCHUNK - 3, buf0, sem0)
        scat(NCHUNK - 3, buf0)
        gstart(NCHUNK - 1, buf0, sem0)
        gwait(NCHUNK - 2, buf1, sem1)
        scat(NCHUNK - 2, buf1)
        gwait(NCHUNK - 1, buf0, sem0)
        scat(NCHUNK - 1, buf0)

        plsc.subcore_barrier()
        for t in range(ROWS_T // 128):
            base = sid * ROWS_T + t * 128
            pltpu.sync_copy(acc_sh.at[pl.ds(base, 128)],
                            out_hbm.at[pl.ds(cid * NPAD + base, 128)])

    return functools.partial(
        pl.kernel,
        out_type=jax.ShapeDtypeStruct((NC * NPAD, DH), jnp.float32),
        mesh=_mesh,
        scratch_types=[
            pltpu.VMEM((NCHUNK, CHUNK), jnp.int32),
            pltpu.VMEM((NCHUNK, CHUNK), jnp.int32),
            pltpu.VMEM((CHUNK, DH), jnp.float32),
            pltpu.VMEM((CHUNK, DH), jnp.float32),
            pltpu.VMEM_SHARED((NPAD, DH), jnp.float32),
            pltpu.SemaphoreType.DMA,
            pltpu.SemaphoreType.DMA,
        ],
        compiler_params=pltpu.CompilerParams(use_tc_tiling_on_sc=False),
    )(body)


_agg_hid = _make_agg(64)
_agg_out = _make_agg(32)


def _feat0_body(x_ref, deg_ref, w_ref, o_ref):
    norm = lax.rsqrt(jnp.maximum(deg_ref[...], 1.0))
    o_ref[0] = jnp.dot(x_ref[...] * norm, w_ref[0],
                       preferred_element_type=jnp.float32)


def _mid_body(p_ref, degi_ref, dego_ref, b0_ref, w1_ref, o_ref):
    agg = jnp.concatenate([p_ref[0], p_ref[1]], axis=1)
    ni = lax.rsqrt(jnp.maximum(degi_ref[...], 1.0))
    h = jnp.maximum(agg * ni + b0_ref[...], 0.0)
    ns = lax.rsqrt(jnp.maximum(dego_ref[...], 1.0))
    o_ref[0] = jnp.dot(h * ns, w1_ref[0],
                       preferred_element_type=jnp.float32)


def _fin_body(p_ref, degi_ref, b1_ref, o_ref):
    agg = jnp.concatenate([p_ref[0], p_ref[1]], axis=1)
    ni = lax.rsqrt(jnp.maximum(degi_ref[...], 1.0))
    o_ref[...] = agg * ni + b1_ref[...]


_BLK = 1024
_GRID = NPAD // _BLK


def _feat0(x_pad, deg, W0):
    return pl.pallas_call(
        _feat0_body,
        grid=(2, _GRID),
        in_specs=[
            pl.BlockSpec((_BLK, 128), lambda j, i: (i, 0)),
            pl.BlockSpec((_BLK, 1), lambda j, i: (i, 0)),
            pl.BlockSpec((1, 128, 64), lambda j, i: (j, 0, 0)),
        ],
        out_specs=pl.BlockSpec((1, _BLK, 64), lambda j, i: (j, i, 0)),
        out_shape=jax.ShapeDtypeStruct((NC, NPAD, 64), jnp.float32),
    )(x_pad, deg, W0)


def _mid(p, degi, dego, b0, W1):
    return pl.pallas_call(
        _mid_body,
        grid=(2, _GRID),
        in_specs=[
            pl.BlockSpec((NC, _BLK, 64), lambda j, i: (0, i, 0)),
            pl.BlockSpec((_BLK, 1), lambda j, i: (i, 0)),
            pl.BlockSpec((_BLK, 1), lambda j, i: (i, 0)),
            pl.BlockSpec((1, 128), lambda j, i: (0, 0)),
            pl.BlockSpec((1, 128, 32), lambda j, i: (j, 0, 0)),
        ],
        out_specs=pl.BlockSpec((1, _BLK, 32), lambda j, i: (j, i, 0)),
        out_shape=jax.ShapeDtypeStruct((NC, NPAD, 32), jnp.float32),
    )(p, degi, dego, b0, W1)


def _fin(p, degi, b1):
    return pl.pallas_call(
        _fin_body,
        grid=(_GRID,),
        in_specs=[
            pl.BlockSpec((NC, _BLK, 32), lambda i: (0, i, 0)),
            pl.BlockSpec((_BLK, 1), lambda i: (i, 0)),
            pl.BlockSpec((1, 64), lambda i: (0, 0)),
        ],
        out_specs=pl.BlockSpec((_BLK, 64), lambda i: (i, 0)),
        out_shape=jax.ShapeDtypeStruct((NPAD, 64), jnp.float32),
    )(p, degi, b1)


def _edge_split(idx, pad_value):
    t = idx.reshape(NS, ET)
    t = jnp.pad(t, ((0, 0), (0, ETP - ET)), constant_values=pad_value)
    return t.reshape(NS, NCHUNK, CHUNK)


def kernel(x, edge_index_l0, edge_index_l1, W0, b0, W1, b1):
    src0 = edge_index_l0[0].astype(jnp.int32)
    dst0 = edge_index_l0[1].astype(jnp.int32)
    src1 = edge_index_l1[0].astype(jnp.int32)
    dst1 = edge_index_l1[1].astype(jnp.int32)

    hist_in = jnp.stack([src0, dst0, src1, dst1]).reshape(
        4 * NS, NHCHUNK, HCHUNK)
    hist = _hist_call(hist_in).reshape(4, NPAD)
    deg_src0 = hist[0].reshape(NPAD, 1)
    deg_dst0 = hist[1].reshape(NPAD, 1)
    deg_src1 = hist[2].reshape(NPAD, 1)
    deg_dst1 = hist[3].reshape(NPAD, 1)

    x_pad = jnp.pad(x, ((0, NPAD - N), (0, 0)))
    W0h = jnp.stack([W0[:, :64], W0[:, 64:]])
    W1h = jnp.stack([W1[:, :32], W1[:, 32:]])
    feat0 = _feat0(x_pad, deg_src0, W0h).reshape(NC * NPAD, 64)

    p0 = _agg_hid(feat0, _edge_split(src0, 0),
                  _edge_split(dst0, N)).reshape(NC, NPAD, 64)
    feat1 = _mid(p0, deg_dst0, deg_src1,
                 b0.reshape(1, 128), W1h).reshape(NC * NPAD, 32)

    p1 = _agg_out(feat1, _edge_split(src1, 0),
                  _edge_split(dst1, N)).reshape(NC, NPAD, 32)
    out = _fin(p1, deg_dst1, b1.reshape(1, 64))

    total_flops = float(
        (2 * 128 * E + 2 * 128 * 128 * N + N * 128) / 1e12
        + (2 * 128 * E + 2 * 128 * 64 * N + N * 128) / 1e12)
    return out[:N], total_flops


# speedup vs baseline: 9.7691x; 1.0052x over previous
import functools

import jax
import jax.numpy as jnp
from jax import lax
from jax.experimental import pallas as pl
from jax.experimental.pallas import tpu as pltpu
from jax.experimental.pallas import tpu_sc as plsc

N = 10000
NPAD = 10240
E = 320000
NC, NS = 2, 16
ET = E // NS
CHUNK = 128
NCHUNK = -(-ET // CHUNK)
ETP = NCHUNK * CHUNK
HCHUNK = 125
NHCHUNK = ET // HCHUNK
ROWS_T = NPAD // NS

_mesh = plsc.VectorSubcoreMesh(
    core_axis_name="c", subcore_axis_name="s", num_cores=NC, num_subcores=NS)


def _hist_body(ei0_hbm, ei1_hbm, out_hbm, idx_v, ones_v, zero_v,
               hist_a, hist_b):
    cid = lax.axis_index("c")
    sid = lax.axis_index("s")

    @pl.loop(0, 8)
    def _(i):
        ones_v[pl.ds(i * 16, 16)] = jnp.ones((16,), jnp.float32)

    @pl.loop(0, ROWS_T // 16)
    def _(i):
        zero_v[pl.ds(i * 16, 16)] = jnp.zeros((16,), jnp.float32)

    pltpu.sync_copy(zero_v, hist_a.at[pl.ds(sid * ROWS_T, ROWS_T)])
    pltpu.sync_copy(zero_v, hist_b.at[pl.ds(sid * ROWS_T, ROWS_T)])
    plsc.subcore_barrier()

    def do_hists(ei_hbm):
        for k, hist in ((0, hist_a), (1, hist_b)):
            m = k * NS + sid
            pltpu.sync_copy(ei_hbm.at[m], idx_v)

            @pl.loop(0, NHCHUNK)
            def _(j):
                pltpu.sync_copy(ones_v.at[pl.ds(0, HCHUNK)],
                                hist.at[idx_v.at[j]], add=True)

    @pl.when(cid == 0)
    def _():
        do_hists(ei0_hbm)

    @pl.when(cid == 1)
    def _():
        do_hists(ei1_hbm)

    plsc.subcore_barrier()
    pltpu.sync_copy(hist_a.at[pl.ds(sid * ROWS_T, ROWS_T)],
                    out_hbm.at[pl.ds((2 * cid) * NPAD + sid * ROWS_T, ROWS_T)])
    pltpu.sync_copy(hist_b.at[pl.ds(sid * ROWS_T, ROWS_T)],
                    out_hbm.at[pl.ds((2 * cid + 1) * NPAD + sid * ROWS_T, ROWS_T)])


_hist_call = functools.partial(
    pl.kernel,
    out_type=jax.ShapeDtypeStruct((4 * NPAD,), jnp.float32),
    mesh=_mesh,
    scratch_types=[
        pltpu.VMEM((NHCHUNK, HCHUNK), jnp.int32),
        pltpu.VMEM((128,), jnp.float32),
        pltpu.VMEM((ROWS_T,), jnp.float32),
        pltpu.VMEM_SHARED((NPAD,), jnp.float32),
        pltpu.VMEM_SHARED((NPAD,), jnp.float32),
    ],
)(_hist_body)


def _make_agg(DH):
    def body(feat_hbm, src_hbm, dst_hbm, out_hbm, src_v, dst_v, buf0, buf1,
             acc_sh, sem0, sem1):
        cid = lax.axis_index("c")
        sid = lax.axis_index("s")

        @pl.loop(0, 128)
        def _(r):
            for c in range(DH // 16):
                buf0[r, pl.ds(c * 16, 16)] = jnp.zeros((16,), jnp.float32)

        for t in range(ROWS_T // 128):
            pltpu.sync_copy(buf0, acc_sh.at[pl.ds(sid * ROWS_T + t * 128, 128)])

        pltpu.sync_copy(src_hbm.at[sid], src_v)
        pltpu.sync_copy(dst_hbm.at[sid], dst_v)

        off = cid * N

        @pl.loop(0, NCHUNK)
        def _(j):
            for c in range(CHUNK // 16):
                sl = pl.ds(c * 16, 16)
                src_v[j, sl] = src_v[j, sl] + off

        plsc.subcore_barrier()

        def gstart(j, buf, sem):
            pltpu.async_copy(feat_hbm.at[src_v.at[j]], buf, sem)

        def gwait(j, buf, sem):
            pltpu.make_async_copy(feat_hbm.at[src_v.at[j]], buf, sem).wait()

        def scat(j, buf):
            pltpu.sync_copy(buf, acc_sh.at[dst_v.at[j]], add=True)

        gstart(0, buf0, sem0)
        gstart(1, buf1, sem1)

        @pl.loop(0, (NCHUNK - 3) // 2)
        def _(i):
            j = 2 * i
            gwait(j, buf0, sem0)
            scat(j, buf0)
            gstart(j + 2, buf0, sem0)
            gwait(j + 1, buf1, sem1)
            scat(j + 1, buf1)
            gstart(j + 3, buf1, sem1)

        gwait(NCHUNK - 3, buf0, sem0)
        scat(NCHUNK - 3, buf0)
        gstart(NCHUNK - 1, buf0, sem0)
        gwait(NCHUNK - 2, buf1, sem1)
        scat(NCHUNK - 2, buf1)
        gwait(NCHUNK - 1, buf0, sem0)
        scat(NCHUNK - 1, buf0)

        plsc.subcore_barrier()
        for t in range(ROWS_T // 128):
            base = sid * ROWS_T + t * 128
            pltpu.sync_copy(acc_sh.at[pl.ds(base, 128)],
                            out_hbm.at[pl.ds(cid * NPAD + base, 128)])

    return functools.partial(
        pl.kernel,
        out_type=jax.ShapeDtypeStruct((NC * NPAD, DH), jnp.float32),
        mesh=_mesh,
        scratch_types=[
            pltpu.VMEM((NCHUNK, CHUNK), jnp.int32),
            pltpu.VMEM((NCHUNK, CHUNK), jnp.int32),
            pltpu.VMEM((CHUNK, DH), jnp.float32),
            pltpu.VMEM((CHUNK, DH), jnp.float32),
            pltpu.VMEM_SHARED((NPAD, DH), jnp.float32),
            pltpu.SemaphoreType.DMA,
            pltpu.SemaphoreType.DMA,
        ],
        compiler_params=pltpu.CompilerParams(use_tc_tiling_on_sc=False),
    )(body)


_agg_hid = _make_agg(64)
_agg_out = _make_agg(32)


def _feat0_body(x_ref, deg_ref, w_ref, o_ref):
    norm = lax.rsqrt(jnp.maximum(deg_ref[...], 1.0))
    o_ref[0] = jnp.dot(x_ref[...] * norm, w_ref[0],
                       preferred_element_type=jnp.float32)


def _mid_body(p_ref, degi_ref, dego_ref, b0_ref, w1_ref, o_ref):
    agg = jnp.concatenate([p_ref[0], p_ref[1]], axis=1)
    ni = lax.rsqrt(jnp.maximum(degi_ref[...], 1.0))
    h = jnp.maximum(agg * ni + b0_ref[...], 0.0)
    ns = lax.rsqrt(jnp.maximum(dego_ref[...], 1.0))
    o_ref[0] = jnp.dot(h * ns, w1_ref[0],
                       preferred_element_type=jnp.float32)


def _fin_body(p_ref, degi_ref, b1_ref, o_ref):
    agg = jnp.concatenate([p_ref[0], p_ref[1]], axis=1)
    ni = lax.rsqrt(jnp.maximum(degi_ref[...], 1.0))
    o_ref[...] = agg * ni + b1_ref[...]


_BLK = 1000
_GRID = N // _BLK


def _feat0(x_pad, deg, W0):
    return pl.pallas_call(
        _feat0_body,
        grid=(2, _GRID),
        in_specs=[
            pl.BlockSpec((_BLK, 128), lambda j, i: (i, 0)),
            pl.BlockSpec((_BLK, 1), lambda j, i: (i, 0)),
            pl.BlockSpec((1, 128, 64), lambda j, i: (j, 0, 0)),
        ],
        out_specs=pl.BlockSpec((1, _BLK, 64), lambda j, i: (j, i, 0)),
        out_shape=jax.ShapeDtypeStruct((NC, N, 64), jnp.float32),
    )(x_pad, deg, W0)


def _mid(p, degi, dego, b0, W1):
    return pl.pallas_call(
        _mid_body,
        grid=(2, _GRID),
        in_specs=[
            pl.BlockSpec((NC, _BLK, 64), lambda j, i: (0, i, 0)),
            pl.BlockSpec((_BLK, 1), lambda j, i: (i, 0)),
            pl.BlockSpec((_BLK, 1), lambda j, i: (i, 0)),
            pl.BlockSpec((1, 128), lambda j, i: (0, 0)),
            pl.BlockSpec((1, 128, 32), lambda j, i: (j, 0, 0)),
        ],
        out_specs=pl.BlockSpec((1, _BLK, 32), lambda j, i: (j, i, 0)),
        out_shape=jax.ShapeDtypeStruct((NC, N, 32), jnp.float32),
    )(p, degi, dego, b0, W1)


def _fin(p, degi, b1):
    return pl.pallas_call(
        _fin_body,
        grid=(_GRID,),
        in_specs=[
            pl.BlockSpec((NC, _BLK, 32), lambda i: (0, i, 0)),
            pl.BlockSpec((_BLK, 1), lambda i: (i, 0)),
            pl.BlockSpec((1, 64), lambda i: (0, 0)),
        ],
        out_specs=pl.BlockSpec((_BLK, 64), lambda i: (i, 0)),
        out_shape=jax.ShapeDtypeStruct((N, 64), jnp.float32),
    )(p, degi, b1)


def _edge_split(idx, pad_value):
    t = idx.reshape(NS, ET)
    t = jnp.pad(t, ((0, 0), (0, ETP - ET)), constant_values=pad_value)
    return t.reshape(NS, NCHUNK, CHUNK)


def kernel(x, edge_index_l0, edge_index_l1, W0, b0, W1, b1):
    ei0 = edge_index_l0.astype(jnp.int32)
    ei1 = edge_index_l1.astype(jnp.int32)
    src0, dst0 = ei0[0], ei0[1]
    src1, dst1 = ei1[0], ei1[1]

    hist = _hist_call(ei0.reshape(2 * NS, NHCHUNK, HCHUNK),
                      ei1.reshape(2 * NS, NHCHUNK, HCHUNK)).reshape(4, NPAD)
    deg_src0 = hist[0].reshape(NPAD, 1)
    deg_dst0 = hist[1].reshape(NPAD, 1)
    deg_src1 = hist[2].reshape(NPAD, 1)
    deg_dst1 = hist[3].reshape(NPAD, 1)

    W0h = jnp.stack([W0[:, :64], W0[:, 64:]])
    W1h = jnp.stack([W1[:, :32], W1[:, 32:]])
    feat0 = _feat0(x, deg_src0, W0h).reshape(NC * N, 64)

    p0 = _agg_hid(feat0, _edge_split(src0, 0),
                  _edge_split(dst0, N)).reshape(NC, NPAD, 64)
    feat1 = _mid(p0, deg_dst0, deg_src1,
                 b0.reshape(1, 128), W1h).reshape(NC * N, 32)

    p1 = _agg_out(feat1, _edge_split(src1, 0),
                  _edge_split(dst1, N)).reshape(NC, NPAD, 32)
    out = _fin(p1, deg_dst1, b1.reshape(1, 64))

    total_flops = float(
        (2 * 128 * E + 2 * 128 * 128 * N + N * 128) / 1e12
        + (2 * 128 * E + 2 * 128 * 64 * N + N * 128) / 1e12)
    return out, total_flops


# speedup vs baseline: 10.3559x; 1.0601x over previous
import functools

import jax
import jax.numpy as jnp
from jax import lax
from jax.experimental import pallas as pl
from jax.experimental.pallas import tpu as pltpu
from jax.experimental.pallas import tpu_sc as plsc

N = 10000
NPAD = 10240
E = 320000
NC, NS = 2, 16
ET = E // NS
CHUNK = 128
NCHUNK = -(-ET // CHUNK)
ETP = NCHUNK * CHUNK
HCHUNK = 125
NHCHUNK = ET // HCHUNK
ROWS_T = NPAD // NS

_mesh = plsc.VectorSubcoreMesh(
    core_axis_name="c", subcore_axis_name="s", num_cores=NC, num_subcores=NS)


def _hist_body(ei0_hbm, ei1_hbm, out_hbm, idx_v, ones_v, zero_v,
               hist_a, hist_b, sem):
    cid = lax.axis_index("c")
    sid = lax.axis_index("s")

    @pl.loop(0, 8)
    def _(i):
        ones_v[pl.ds(i * 16, 16)] = jnp.ones((16,), jnp.float32)

    @pl.loop(0, ROWS_T // 16)
    def _(i):
        zero_v[pl.ds(i * 16, 16)] = jnp.zeros((16,), jnp.float32)

    pltpu.sync_copy(zero_v, hist_a.at[pl.ds(sid * ROWS_T, ROWS_T)])
    pltpu.sync_copy(zero_v, hist_b.at[pl.ds(sid * ROWS_T, ROWS_T)])
    plsc.subcore_barrier()

    def do_hists(ei_hbm, sem):
        for k, hist in ((0, hist_a), (1, hist_b)):
            m = k * NS + sid
            pltpu.sync_copy(ei_hbm.at[m], idx_v)

            @pl.loop(0, NHCHUNK // 8)
            def _(i):
                for b in range(8):
                    pltpu.async_copy(ones_v.at[pl.ds(0, HCHUNK)],
                                     hist.at[idx_v.at[8 * i + b]], sem,
                                     add=True)
                for b in range(8):
                    pltpu.make_async_copy(
                        ones_v.at[pl.ds(0, HCHUNK)],
                        hist.at[idx_v.at[8 * i + b]], sem).wait()

    @pl.when(cid == 0)
    def _():
        do_hists(ei0_hbm, sem)

    @pl.when(cid == 1)
    def _():
        do_hists(ei1_hbm, sem)

    plsc.subcore_barrier()
    pltpu.sync_copy(hist_a.at[pl.ds(sid * ROWS_T, ROWS_T)],
                    out_hbm.at[pl.ds((2 * cid) * NPAD + sid * ROWS_T, ROWS_T)])
    pltpu.sync_copy(hist_b.at[pl.ds(sid * ROWS_T, ROWS_T)],
                    out_hbm.at[pl.ds((2 * cid + 1) * NPAD + sid * ROWS_T, ROWS_T)])


_hist_call = functools.partial(
    pl.kernel,
    out_type=jax.ShapeDtypeStruct((4 * NPAD,), jnp.float32),
    mesh=_mesh,
    scratch_types=[
        pltpu.VMEM((NHCHUNK, HCHUNK), jnp.int32),
        pltpu.VMEM((128,), jnp.float32),
        pltpu.VMEM((ROWS_T,), jnp.float32),
        pltpu.VMEM_SHARED((NPAD,), jnp.float32),
        pltpu.VMEM_SHARED((NPAD,), jnp.float32),
        pltpu.SemaphoreType.DMA,
    ],
)(_hist_body)


def _make_agg(DH, fused_fin=False):
    def body(feat_hbm, src_hbm, dst_hbm, *refs):
        if fused_fin:
            (norm_hbm, bias_hbm, out_hbm, src_v, dst_v,
             buf0, buf1, buf2, buf3, wbuf, norm_v, bias_v, acc_sh,
             sg0, sg1, sg2, sg3, ss0, ss1, ss2, ss3) = refs
        else:
            (out_hbm, src_v, dst_v,
             buf0, buf1, buf2, buf3, acc_sh,
             sg0, sg1, sg2, sg3, ss0, ss1, ss2, ss3) = refs
        cid = lax.axis_index("c")
        sid = lax.axis_index("s")
        bufs = (buf0, buf1, buf2, buf3)
        sgs = (sg0, sg1, sg2, sg3)
        sss = (ss0, ss1, ss2, ss3)

        @pl.loop(0, 128)
        def _(r):
            for c in range(DH // 16):
                buf0[r, pl.ds(c * 16, 16)] = jnp.zeros((16,), jnp.float32)

        for t in range(ROWS_T // 128):
            pltpu.sync_copy(buf0, acc_sh.at[pl.ds(sid * ROWS_T + t * 128, 128)])

        pltpu.sync_copy(src_hbm.at[sid], src_v)
        pltpu.sync_copy(dst_hbm.at[sid], dst_v)

        off = cid * N

        @pl.loop(0, NCHUNK)
        def _(j):
            for c in range(CHUNK // 16):
                sl = pl.ds(c * 16, 16)
                src_v[j, sl] = src_v[j, sl] + off

        plsc.subcore_barrier()

        def gstart(j, b):
            pltpu.async_copy(feat_hbm.at[src_v.at[j]], bufs[b], sgs[b])

        def gwait(j, b):
            pltpu.make_async_copy(
                feat_hbm.at[src_v.at[j]], bufs[b], sgs[b]).wait()

        def sstart(j, b):
            pltpu.async_copy(bufs[b], acc_sh.at[dst_v.at[j]], sss[b],
                             add=True)

        def swait(j, b):
            pltpu.make_async_copy(
                bufs[b], acc_sh.at[dst_v.at[j]], sss[b]).wait()

        def step(j, b, do_swait, do_gstart):
            gwait(j, b)
            sstart(j, b)
            if do_swait:
                swait(j - 2, (b + 2) % 4)
            if do_gstart:
                gstart(j + 2, (b + 2) % 4)

        gstart(0, 0)
        gstart(1, 1)
        step(0, 0, False, True)
        step(1, 1, False, True)
        step(2, 2, True, True)
        step(3, 3, True, True)

        @pl.loop(1, (NCHUNK - 9) // 4 + 1)
        def _(i):
            j = 4 * i
            for r in range(4):
                step(j + r, r, True, True)

        for j, b, dw, dg in ((NCHUNK - 5, 0, True, True),
                             (NCHUNK - 4, 1, True, True),
                             (NCHUNK - 3, 2, True, True),
                             (NCHUNK - 2, 3, True, False),
                             (NCHUNK - 1, 0, False, False)):
            step(j, b, dw, dg)
        swait(NCHUNK - 3, 2)
        swait(NCHUNK - 2, 3)
        swait(NCHUNK - 1, 0)

        plsc.subcore_barrier()
        if not fused_fin:
            for t in range(ROWS_T // 128):
                base = sid * ROWS_T + t * 128
                pltpu.sync_copy(acc_sh.at[pl.ds(base, 128)],
                                out_hbm.at[pl.ds(cid * NPAD + base, 128)])
        else:
            pltpu.sync_copy(bias_hbm.at[cid], bias_v)

            @pl.when(sid < NS - 1)
            def _():
                pltpu.sync_copy(norm_hbm.at[pl.ds(sid * ROWS_T, ROWS_T)],
                                norm_v)

            @pl.when(sid == NS - 1)
            def _():
                pltpu.sync_copy(norm_hbm.at[pl.ds((NS - 1) * ROWS_T,
                                                  N - (NS - 1) * ROWS_T)],
                                norm_v.at[pl.ds(0, N - (NS - 1) * ROWS_T)])

            def wblock(t, nr):
                s0 = sid * ROWS_T + t * 128
                pltpu.sync_copy(acc_sh.at[pl.ds(s0, nr)],
                                wbuf.at[pl.ds(0, nr)])

                @pl.loop(0, nr // 16)
                def _(g):
                    nv = norm_v[pl.ds(t * 128 + g * 16, 16)]
                    for k in range(16):
                        r = g * 16 + k
                        s = nv[k]
                        for c in range(DH // 16):
                            sl = pl.ds(c * 16, 16)
                            wbuf[r, sl] = wbuf[r, sl] * s + bias_v[sl]

                pltpu.sync_copy(wbuf.at[pl.ds(0, nr)],
                                out_hbm.at[cid, pl.ds(s0, nr)])

            full_blocks = (N - (NS - 1) * ROWS_T) // 128
            tail_rows = N - (NS - 1) * ROWS_T - full_blocks * 128

            @pl.when(sid < NS - 1)
            def _():
                for t in range(ROWS_T // 128):
                    wblock(t, 128)

            @pl.when(sid == NS - 1)
            def _():
                for t in range(full_blocks):
                    wblock(t, 128)
                wblock(full_blocks, tail_rows)

    if fused_fin:
        out_type = jax.ShapeDtypeStruct((NC, N, DH), jnp.float32)
        extra = [pltpu.VMEM((128, DH), jnp.float32),
                 pltpu.VMEM((ROWS_T,), jnp.float32),
                 pltpu.VMEM((DH,), jnp.float32)]
    else:
        out_type = jax.ShapeDtypeStruct((NC * NPAD, DH), jnp.float32)
        extra = []
    return functools.partial(
        pl.kernel,
        out_type=out_type,
        mesh=_mesh,
        scratch_types=[
            pltpu.VMEM((NCHUNK, CHUNK), jnp.int32),
            pltpu.VMEM((NCHUNK, CHUNK), jnp.int32),
            pltpu.VMEM((CHUNK, DH), jnp.float32),
            pltpu.VMEM((CHUNK, DH), jnp.float32),
            pltpu.VMEM((CHUNK, DH), jnp.float32),
            pltpu.VMEM((CHUNK, DH), jnp.float32),
        ] + extra + [
            pltpu.VMEM_SHARED((NPAD, DH), jnp.float32),
        ] + [pltpu.SemaphoreType.DMA] * 8,
        compiler_params=pltpu.CompilerParams(use_tc_tiling_on_sc=False),
    )(body)


_agg_hid = _make_agg(64)
_agg_out = _make_agg(32, fused_fin=True)


def _feat0_body(x_ref, deg_ref, degd1_ref, w_ref, o_ref, n1_ref):
    norm = lax.rsqrt(jnp.maximum(deg_ref[...], 1.0))
    o_ref[0] = jnp.dot(x_ref[...] * norm, w_ref[0],
                       preferred_element_type=jnp.float32)
    n1_ref[...] = lax.rsqrt(jnp.maximum(degd1_ref[...], 1.0))


def _mid_body(p_ref, degi_ref, dego_ref, b0_ref, w1_ref, o_ref):
    agg = jnp.concatenate([p_ref[0], p_ref[1]], axis=1)
    ni = lax.rsqrt(jnp.maximum(degi_ref[...], 1.0))
    h = jnp.maximum(agg * ni + b0_ref[...], 0.0)
    ns = lax.rsqrt(jnp.maximum(dego_ref[...], 1.0))
    o_ref[0] = jnp.dot(h * ns, w1_ref[0],
                       preferred_element_type=jnp.float32)


_BLK = 1000
_GRID = N // _BLK


def _feat0(x_pad, deg, degd1, W0):
    return pl.pallas_call(
        _feat0_body,
        grid=(2, _GRID),
        in_specs=[
            pl.BlockSpec((_BLK, 128), lambda j, i: (i, 0)),
            pl.BlockSpec((_BLK, 1), lambda j, i: (i, 0)),
            pl.BlockSpec((_BLK, 1), lambda j, i: (i, 0)),
            pl.BlockSpec((1, 128, 64), lambda j, i: (j, 0, 0)),
        ],
        out_specs=[
            pl.BlockSpec((1, _BLK, 64), lambda j, i: (j, i, 0)),
            pl.BlockSpec((_BLK, 1), lambda j, i: (i, 0)),
        ],
        out_shape=[
            jax.ShapeDtypeStruct((NC, N, 64), jnp.float32),
            jax.ShapeDtypeStruct((N, 1), jnp.float32),
        ],
    )(x_pad, deg, degd1, W0)


def _mid(p, degi, dego, b0, W1):
    return pl.pallas_call(
        _mid_body,
        grid=(2, _GRID),
        in_specs=[
            pl.BlockSpec((NC, _BLK, 64), lambda j, i: (0, i, 0)),
            pl.BlockSpec((_BLK, 1), lambda j, i: (i, 0)),
            pl.BlockSpec((_BLK, 1), lambda j, i: (i, 0)),
            pl.BlockSpec((1, 128), lambda j, i: (0, 0)),
            pl.BlockSpec((1, 128, 32), lambda j, i: (j, 0, 0)),
        ],
        out_specs=pl.BlockSpec((1, _BLK, 32), lambda j, i: (j, i, 0)),
        out_shape=jax.ShapeDtypeStruct((NC, N, 32), jnp.float32),
    )(p, degi, dego, b0, W1)


def _edge_split(idx, pad_value):
    t = idx.reshape(NS, ET)
    t = jnp.pad(t, ((0, 0), (0, ETP - ET)), constant_values=pad_value)
    return t.reshape(NS, NCHUNK, CHUNK)


def kernel(x, edge_index_l0, edge_index_l1, W0, b0, W1, b1):
    ei0 = edge_index_l0.astype(jnp.int32)
    ei1 = edge_index_l1.astype(jnp.int32)
    src0, dst0 = ei0[0], ei0[1]
    src1, dst1 = ei1[0], ei1[1]

    hist = _hist_call(ei0.reshape(2 * NS, NHCHUNK, HCHUNK),
                      ei1.reshape(2 * NS, NHCHUNK, HCHUNK)).reshape(4, NPAD)
    deg_src0 = hist[0].reshape(NPAD, 1)
    deg_dst0 = hist[1].reshape(NPAD, 1)
    deg_src1 = hist[2].reshape(NPAD, 1)
    deg_dst1 = hist[3].reshape(NPAD, 1)

    W0h = jnp.stack([W0[:, :64], W0[:, 64:]])
    W1h = jnp.stack([W1[:, :32], W1[:, 32:]])
    b1h = jnp.stack([b1[:32], b1[32:]])
    feat0, norm1 = _feat0(x, deg_src0, deg_dst1, W0h)
    feat0 = feat0.reshape(NC * N, 64)

    p0 = _agg_hid(feat0, _edge_split(src0, 0),
                  _edge_split(dst0, N)).reshape(NC, NPAD, 64)
    feat1 = _mid(p0, deg_dst0, deg_src1,
                 b0.reshape(1, 128), W1h).reshape(NC * N, 32)

    halves = _agg_out(feat1, _edge_split(src1, 0), _edge_split(dst1, N),
                      norm1.reshape(N), b1h)
    out = jnp.swapaxes(halves, 0, 1).reshape(N, 64)

    total_flops = float(
        (2 * 128 * E + 2 * 128 * 128 * N + N * 128) / 1e12
        + (2 * 128 * E + 2 * 128 * 64 * N + N * 128) / 1e12)
    return out, total_flops


# speedup vs baseline: 10.5490x; 1.0186x over previous
import functools

import jax
import jax.numpy as jnp
from jax import lax
from jax.experimental import pallas as pl
from jax.experimental.pallas import tpu as pltpu
from jax.experimental.pallas import tpu_sc as plsc

N = 10000
NPAD = 10240
NF = N + 8
E = 320000
NC, NS = 2, 16
ET = E // NS
CHUNK = 128
NCHUNK = -(-ET // CHUNK)
ETP = NCHUNK * CHUNK
ROWS_T = NPAD // NS

_mesh = plsc.VectorSubcoreMesh(
    core_axis_name="c", subcore_axis_name="s", num_cores=NC, num_subcores=NS)


def _hist_body(s0_hbm, d0_hbm, s1_hbm, d1_hbm, out_hbm,
               idx_v, ones_v, zero_v, hist_a, hist_b, sem):
    cid = lax.axis_index("c")
    sid = lax.axis_index("s")

    @pl.loop(0, 8)
    def _(i):
        ones_v[pl.ds(i * 16, 16)] = jnp.ones((16,), jnp.float32)

    @pl.loop(0, ROWS_T // 16)
    def _(i):
        zero_v[pl.ds(i * 16, 16)] = jnp.zeros((16,), jnp.float32)

    pltpu.sync_copy(zero_v, hist_a.at[pl.ds(sid * ROWS_T, ROWS_T)])
    pltpu.sync_copy(zero_v, hist_b.at[pl.ds(sid * ROWS_T, ROWS_T)])
    plsc.subcore_barrier()

    def do_hists(src_hbm, dst_hbm):
        for arr, hist in ((src_hbm, hist_a), (dst_hbm, hist_b)):
            pltpu.sync_copy(arr.at[sid], idx_v)

            @pl.loop(0, NCHUNK)
            def _(j):
                pltpu.async_copy(ones_v, hist.at[idx_v.at[j]], sem, add=True)

                @pl.when(j >= 8)
                def _():
                    pltpu.make_async_copy(
                        ones_v, hist.at[idx_v.at[j - 8]], sem).wait()

            for t in range(8):
                pltpu.make_async_copy(
                    ones_v, hist.at[idx_v.at[NCHUNK - 8 + t]], sem).wait()

    @pl.when(cid == 0)
    def _():
        do_hists(s0_hbm, d0_hbm)

    @pl.when(cid == 1)
    def _():
        do_hists(s1_hbm, d1_hbm)

    plsc.subcore_barrier()
    pltpu.sync_copy(hist_a.at[pl.ds(sid * ROWS_T, ROWS_T)],
                    out_hbm.at[pl.ds((2 * cid) * NPAD + sid * ROWS_T, ROWS_T)])
    pltpu.sync_copy(hist_b.at[pl.ds(sid * ROWS_T, ROWS_T)],
                    out_hbm.at[pl.ds((2 * cid + 1) * NPAD + sid * ROWS_T, ROWS_T)])


_hist_call = functools.partial(
    pl.kernel,
    out_type=jax.ShapeDtypeStruct((4 * NPAD,), jnp.float32),
    mesh=_mesh,
    scratch_types=[
        pltpu.VMEM((NCHUNK, CHUNK), jnp.int32),
        pltpu.VMEM((CHUNK,), jnp.float32),
        pltpu.VMEM((ROWS_T,), jnp.float32),
        pltpu.VMEM_SHARED((NPAD,), jnp.float32),
        pltpu.VMEM_SHARED((NPAD,), jnp.float32),
        pltpu.SemaphoreType.DMA,
    ],
)(_hist_body)


def _make_agg(DH, fused_fin=False):
    def body(feat_lo, feat_hi, src_hbm, dst_hbm, *refs):
        if fused_fin:
            (norm_hbm, bias_hbm, out_hbm, src_v, dst_v,
             buf0, buf1, buf2, buf3, wbuf, norm_v, bias_v, acc_sh,
             sg0, sg1, sg2, sg3, ss0, ss1, ss2, ss3) = refs
        else:
            (out_lo, out_hi, src_v, dst_v,
             buf0, buf1, buf2, buf3, acc_sh,
             sg0, sg1, sg2, sg3, ss0, ss1, ss2, ss3) = refs
        cid = lax.axis_index("c")
        sid = lax.axis_index("s")
        bufs = (buf0, buf1, buf2, buf3)
        sgs = (sg0, sg1, sg2, sg3)
        sss = (ss0, ss1, ss2, ss3)

        @pl.loop(0, 128)
        def _(r):
            for c in range(DH // 16):
                buf0[r, pl.ds(c * 16, 16)] = jnp.zeros((16,), jnp.float32)

        for t in range(ROWS_T // 128):
            pltpu.sync_copy(buf0, acc_sh.at[pl.ds(sid * ROWS_T + t * 128, 128)])

        pltpu.sync_copy(src_hbm.at[sid], src_v)
        pltpu.sync_copy(dst_hbm.at[sid], dst_v)
        plsc.subcore_barrier()

        def ring(feat_hbm):
            def gstart(j, b):
                pltpu.async_copy(feat_hbm.at[src_v.at[j]], bufs[b], sgs[b])

            def gwait(j, b):
                pltpu.make_async_copy(
                    feat_hbm.at[src_v.at[j]], bufs[b], sgs[b]).wait()

            def sstart(j, b):
                pltpu.async_copy(bufs[b], acc_sh.at[dst_v.at[j]], sss[b],
                                 add=True)

            def swait(j, b):
                pltpu.make_async_copy(
                    bufs[b], acc_sh.at[dst_v.at[j]], sss[b]).wait()

            def step(j, b, do_swait, do_gstart):
                gwait(j, b)
                sstart(j, b)
                if do_swait:
                    swait(j - 2, (b + 2) % 4)
                if do_gstart:
                    gstart(j + 2, (b + 2) % 4)

            gstart(0, 0)
            gstart(1, 1)
            step(0, 0, False, True)
            step(1, 1, False, True)
            step(2, 2, True, True)
            step(3, 3, True, True)

            @pl.loop(1, (NCHUNK - 9) // 4 + 1)
            def _(i):
                j = 4 * i
                for r in range(4):
                    step(j + r, r, True, True)

            for j, b, dw, dg in ((NCHUNK - 5, 0, True, True),
                                 (NCHUNK - 4, 1, True, True),
                                 (NCHUNK - 3, 2, True, True),
                                 (NCHUNK - 2, 3, True, False),
                                 (NCHUNK - 1, 0, False, False)):
                step(j, b, dw, dg)
            swait(NCHUNK - 3, 2)
            swait(NCHUNK - 2, 3)
            swait(NCHUNK - 1, 0)

        @pl.when(cid == 0)
        def _():
            ring(feat_lo)

        @pl.when(cid == 1)
        def _():
            ring(feat_hi)

        plsc.subcore_barrier()
        if not fused_fin:
            def wout(out_hbm):
                for t in range(ROWS_T // 128):
                    base = sid * ROWS_T + t * 128
                    pltpu.sync_copy(acc_sh.at[pl.ds(base, 128)],
                                    out_hbm.at[pl.ds(base, 128)])

            @pl.when(cid == 0)
            def _():
                wout(out_lo)

            @pl.when(cid == 1)
            def _():
                wout(out_hi)
        else:
            pltpu.sync_copy(bias_hbm.at[cid], bias_v)

            @pl.when(sid < NS - 1)
            def _():
                pltpu.sync_copy(norm_hbm.at[pl.ds(sid * ROWS_T, ROWS_T)],
                                norm_v)

            @pl.when(sid == NS - 1)
            def _():
                pltpu.sync_copy(norm_hbm.at[pl.ds((NS - 1) * ROWS_T,
                                                  N - (NS - 1) * ROWS_T)],
                                norm_v.at[pl.ds(0, N - (NS - 1) * ROWS_T)])

            def wblock(t, nr):
                s0 = sid * ROWS_T + t * 128
                pltpu.sync_copy(acc_sh.at[pl.ds(s0, nr)],
                                wbuf.at[pl.ds(0, nr)])

                @pl.loop(0, nr // 16)
                def _(g):
                    nv = norm_v[pl.ds(t * 128 + g * 16, 16)]
                    for k in range(16):
                        r = g * 16 + k
                        s = nv[k]
                        for c in range(DH // 16):
                            sl = pl.ds(c * 16, 16)
                            wbuf[r, sl] = wbuf[r, sl] * s + bias_v[sl]

                pltpu.sync_copy(wbuf.at[pl.ds(0, nr)],
                                out_hbm.at[pl.ds(s0, nr),
                                           pl.ds(cid * DH, DH)])

            full_blocks = (N - (NS - 1) * ROWS_T) // 128
            tail_rows = N - (NS - 1) * ROWS_T - full_blocks * 128

            @pl.when(sid < NS - 1)
            def _():
                for t in range(ROWS_T // 128):
                    wblock(t, 128)

            @pl.when(sid == NS - 1)
            def _():
                for t in range(full_blocks):
                    wblock(t, 128)
                wblock(full_blocks, tail_rows)

    if fused_fin:
        out_type = jax.ShapeDtypeStruct((N, 2 * DH), jnp.float32)
        extra = [pltpu.VMEM((128, DH), jnp.float32),
                 pltpu.VMEM((ROWS_T,), jnp.float32),
                 pltpu.VMEM((DH,), jnp.float32)]
    else:
        out_type = [jax.ShapeDtypeStruct((NPAD, DH), jnp.float32),
                    jax.ShapeDtypeStruct((NPAD, DH), jnp.float32)]
        extra = []
    return functools.partial(
        pl.kernel,
        out_type=out_type,
        mesh=_mesh,
        scratch_types=[
            pltpu.VMEM((NCHUNK, CHUNK), jnp.int32),
            pltpu.VMEM((NCHUNK, CHUNK), jnp.int32),
            pltpu.VMEM((CHUNK, DH), jnp.float32),
            pltpu.VMEM((CHUNK, DH), jnp.float32),
            pltpu.VMEM((CHUNK, DH), jnp.float32),
            pltpu.VMEM((CHUNK, DH), jnp.float32),
        ] + extra + [
            pltpu.VMEM_SHARED((NPAD, DH), jnp.float32),
        ] + [pltpu.SemaphoreType.DMA] * 8,
        compiler_params=pltpu.CompilerParams(use_tc_tiling_on_sc=False),
    )(body)


_agg_hid = _make_agg(64)
_agg_out = _make_agg(32, fused_fin=True)


def _feat0_body(x_ref, deg_ref, degd1_ref, w_ref, lo_ref, hi_ref, n1_ref):
    norm = lax.rsqrt(jnp.maximum(deg_ref[...], 1.0))
    res = jnp.dot(x_ref[...] * norm, w_ref[...],
                  preferred_element_type=jnp.float32)
    lo_ref[...] = res[:, :64]
    hi_ref[...] = res[:, 64:]
    n1_ref[...] = lax.rsqrt(jnp.maximum(degd1_ref[...], 1.0))


def _mid_body(plo_ref, phi_ref, degi_ref, dego_ref, b0_ref, w1_ref,
              lo_ref, hi_ref):
    agg = jnp.concatenate([plo_ref[...], phi_ref[...]], axis=1)
    ni = lax.rsqrt(jnp.maximum(degi_ref[...], 1.0))
    h = jnp.maximum(agg * ni + b0_ref[...], 0.0)
    ns = lax.rsqrt(jnp.maximum(dego_ref[...], 1.0))
    res = jnp.dot(h * ns, w1_ref[...], preferred_element_type=jnp.float32)
    lo_ref[...] = res[:, :32]
    hi_ref[...] = res[:, 32:]


_BLK = 1000
_GRID = N // _BLK


def _feat0(x, deg, degd1, W0):
    return pl.pallas_call(
        _feat0_body,
        grid=(_GRID,),
        in_specs=[
            pl.BlockSpec((_BLK, 128), lambda i: (i, 0)),
            pl.BlockSpec((_BLK, 1), lambda i: (i, 0)),
            pl.BlockSpec((_BLK, 1), lambda i: (i, 0)),
            pl.BlockSpec((128, 128), lambda i: (0, 0)),
        ],
        out_specs=[
            pl.BlockSpec((_BLK, 64), lambda i: (i, 0)),
            pl.BlockSpec((_BLK, 64), lambda i: (i, 0)),
            pl.BlockSpec((_BLK, 1), lambda i: (i, 0)),
        ],
        out_shape=[
            jax.ShapeDtypeStruct((NF, 64), jnp.float32),
            jax.ShapeDtypeStruct((NF, 64), jnp.float32),
            jax.ShapeDtypeStruct((N, 1), jnp.float32),
        ],
    )(x, deg, degd1, W0)


def _mid(p_lo, p_hi, degi, dego, b0, W1):
    return pl.pallas_call(
        _mid_body,
        grid=(_GRID,),
        in_specs=[
            pl.BlockSpec((_BLK, 64), lambda i: (i, 0)),
            pl.BlockSpec((_BLK, 64), lambda i: (i, 0)),
            pl.BlockSpec((_BLK, 1), lambda i: (i, 0)),
            pl.BlockSpec((_BLK, 1), lambda i: (i, 0)),
            pl.BlockSpec((1, 128), lambda i: (0, 0)),
            pl.BlockSpec((128, 64), lambda i: (0, 0)),
        ],
        out_specs=[
            pl.BlockSpec((_BLK, 32), lambda i: (i, 0)),
            pl.BlockSpec((_BLK, 32), lambda i: (i, 0)),
        ],
        out_shape=[
            jax.ShapeDtypeStruct((NF, 32), jnp.float32),
            jax.ShapeDtypeStruct((NF, 32), jnp.float32),
        ],
    )(p_lo, p_hi, degi, dego, b0, W1)


def _edge_split(idx):
    t = idx.reshape(NS, ET)
    t = jnp.pad(t, ((0, 0), (0, ETP - ET)), constant_values=N)
    return t.reshape(NS, NCHUNK, CHUNK)


def kernel(x, edge_index_l0, edge_index_l1, W0, b0, W1, b1):
    ei0 = edge_index_l0.astype(jnp.int32)
    ei1 = edge_index_l1.astype(jnp.int32)
    es0 = _edge_split(ei0[0])
    ed0 = _edge_split(ei0[1])
    es1 = _edge_split(ei1[0])
    ed1 = _edge_split(ei1[1])

    hist = _hist_call(es0, ed0, es1, ed1).reshape(4, NPAD)
    deg_src0 = hist[0].reshape(NPAD, 1)
    deg_dst0 = hist[1].reshape(NPAD, 1)
    deg_src1 = hist[2].reshape(NPAD, 1)
    deg_dst1 = hist[3].reshape(NPAD, 1)

    b1h = jnp.stack([b1[:32], b1[32:]])
    f0_lo, f0_hi, norm1 = _feat0(x, deg_src0, deg_dst1, W0)

    p0_lo, p0_hi = _agg_hid(f0_lo, f0_hi, es0, ed0)
    f1_lo, f1_hi = _mid(p0_lo, p0_hi, deg_dst0, deg_src1,
                        b0.reshape(1, 128), W1)

    out = _agg_out(f1_lo, f1_hi, es1, ed1, norm1.reshape(N), b1h)

    total_flops = float(
        (2 * 128 * E + 2 * 128 * 128 * N + N * 128) / 1e12
        + (2 * 128 * E + 2 * 128 * 64 * N + N * 128) / 1e12)
    return out, total_flops


# speedup vs baseline: 11.6856x; 1.1077x over previous
import functools

import jax
import jax.numpy as jnp
from jax import lax
from jax.experimental import pallas as pl
from jax.experimental.pallas import tpu as pltpu
from jax.experimental.pallas import tpu_sc as plsc

N = 10000
NPAD = 10240
NF = N + 8
E = 320000
NC, NS = 2, 16
ET = E // NS
CHUNK = 128
NCHUNK = -(-ET // CHUNK)
ETP = NCHUNK * CHUNK
ROWS_T = NPAD // NS

_mesh = plsc.VectorSubcoreMesh(
    core_axis_name="c", subcore_axis_name="s", num_cores=NC, num_subcores=NS)


def _hist_body(ei0_hbm, ei1_hbm, out_hbm,
               idx_v, ones_v, zero_v, hist_a, hist_b, sem):
    cid = lax.axis_index("c")
    sid = lax.axis_index("s")

    @pl.loop(0, 8)
    def _(i):
        ones_v[pl.ds(i * 16, 16)] = jnp.ones((16,), jnp.float32)

    @pl.loop(0, ROWS_T // 16)
    def _(i):
        zero_v[pl.ds(i * 16, 16)] = jnp.zeros((16,), jnp.float32)

    pltpu.sync_copy(zero_v, hist_a.at[pl.ds(sid * ROWS_T, ROWS_T)])
    pltpu.sync_copy(zero_v, hist_b.at[pl.ds(sid * ROWS_T, ROWS_T)])
    plsc.subcore_barrier()

    def do_hists(ei_hbm):
        for k, hist in ((0, hist_a), (1, hist_b)):
            pltpu.sync_copy(ei_hbm.at[k, pl.ds(sid * ET, ET)],
                            idx_v.at[pl.ds(0, ET)])
            for t in range((ETP - ET) // 16):
                idx_v[pl.ds(ET + 16 * t, 16)] = jnp.full((16,), N, jnp.int32)

            def chunk(j):
                return idx_v.at[pl.ds(j * CHUNK, CHUNK)]

            @pl.loop(0, NCHUNK)
            def _(j):
                pltpu.async_copy(ones_v, hist.at[chunk(j)], sem, add=True)

                @pl.when(j >= 8)
                def _():
                    pltpu.make_async_copy(
                        ones_v, hist.at[chunk(j - 8)], sem).wait()

            for t in range(8):
                pltpu.make_async_copy(
                    ones_v, hist.at[chunk(NCHUNK - 8 + t)], sem).wait()

    @pl.when(cid == 0)
    def _():
        do_hists(ei0_hbm)

    @pl.when(cid == 1)
    def _():
        do_hists(ei1_hbm)

    plsc.subcore_barrier()
    pltpu.sync_copy(hist_a.at[pl.ds(sid * ROWS_T, ROWS_T)],
                    out_hbm.at[pl.ds((2 * cid) * NPAD + sid * ROWS_T, ROWS_T)])
    pltpu.sync_copy(hist_b.at[pl.ds(sid * ROWS_T, ROWS_T)],
                    out_hbm.at[pl.ds((2 * cid + 1) * NPAD + sid * ROWS_T, ROWS_T)])


_hist_call = functools.partial(
    pl.kernel,
    out_type=jax.ShapeDtypeStruct((4 * NPAD,), jnp.float32),
    mesh=_mesh,
    scratch_types=[
        pltpu.VMEM((ETP,), jnp.int32),
        pltpu.VMEM((CHUNK,), jnp.float32),
        pltpu.VMEM((ROWS_T,), jnp.float32),
        pltpu.VMEM_SHARED((NPAD,), jnp.float32),
        pltpu.VMEM_SHARED((NPAD,), jnp.float32),
        pltpu.SemaphoreType.DMA,
    ],
    compiler_params=pltpu.CompilerParams(use_tc_tiling_on_sc=False),
)(_hist_body)


def _make_agg(DH, fused_fin=False):
    def body(feat_lo, feat_hi, ei_hbm, *refs):
        if fused_fin:
            (norm_hbm, bias_hbm, out_hbm, src_v, dst_v,
             buf0, buf1, buf2, buf3, wbuf, norm_v, bias_v, acc_sh,
             sg0, sg1, sg2, sg3, ss0, ss1, ss2, ss3) = refs
        else:
            (out_lo, out_hi, src_v, dst_v,
             buf0, buf1, buf2, buf3, acc_sh,
             sg0, sg1, sg2, sg3, ss0, ss1, ss2, ss3) = refs
        cid = lax.axis_index("c")
        sid = lax.axis_index("s")
        bufs = (buf0, buf1, buf2, buf3)
        sgs = (sg0, sg1, sg2, sg3)
        sss = (ss0, ss1, ss2, ss3)

        @pl.loop(0, 128)
        def _(r):
            for c in range(DH // 16):
                buf0[r, pl.ds(c * 16, 16)] = jnp.zeros((16,), jnp.float32)

        for t in range(ROWS_T // 128):
            pltpu.sync_copy(buf0, acc_sh.at[pl.ds(sid * ROWS_T + t * 128, 128)])

        pltpu.sync_copy(ei_hbm.at[0, pl.ds(sid * ET, ET)],
                        src_v.at[pl.ds(0, ET)])
        pltpu.sync_copy(ei_hbm.at[1, pl.ds(sid * ET, ET)],
                        dst_v.at[pl.ds(0, ET)])
        for t in range((ETP - ET) // 16):
            sl = pl.ds(ET + 16 * t, 16)
            src_v[sl] = jnp.full((16,), N, jnp.int32)
            dst_v[sl] = jnp.full((16,), N, jnp.int32)

        def schunk(j):
            return src_v.at[pl.ds(j * CHUNK, CHUNK)]

        def dchunk(j):
            return dst_v.at[pl.ds(j * CHUNK, CHUNK)]

        plsc.subcore_barrier()

        def ring(feat_hbm):
            def gstart(j, b):
                pltpu.async_copy(feat_hbm.at[schunk(j)], bufs[b], sgs[b])

            def gwait(j, b):
                pltpu.make_async_copy(
                    feat_hbm.at[schunk(j)], bufs[b], sgs[b]).wait()

            def sstart(j, b):
                pltpu.async_copy(bufs[b], acc_sh.at[dchunk(j)], sss[b],
                                 add=True)

            def swait(j, b):
                pltpu.make_async_copy(
                    bufs[b], acc_sh.at[dchunk(j)], sss[b]).wait()

            def step(j, b, do_swait, do_gstart):
                gwait(j, b)
                sstart(j, b)
                if do_swait:
                    swait(j - 2, (b + 2) % 4)
                if do_gstart:
                    gstart(j + 2, (b + 2) % 4)

            gstart(0, 0)
            gstart(1, 1)
            step(0, 0, False, True)
            step(1, 1, False, True)
            step(2, 2, True, True)
            step(3, 3, True, True)

            @pl.loop(1, (NCHUNK - 9) // 4 + 1)
            def _(i):
                j = 4 * i
                for r in range(4):
                    step(j + r, r, True, True)

            for j, b, dw, dg in ((NCHUNK - 5, 0, True, True),
                                 (NCHUNK - 4, 1, True, True),
                                 (NCHUNK - 3, 2, True, True),
                                 (NCHUNK - 2, 3, True, False),
                                 (NCHUNK - 1, 0, False, False)):
                step(j, b, dw, dg)
            swait(NCHUNK - 3, 2)
            swait(NCHUNK - 2, 3)
            swait(NCHUNK - 1, 0)

        @pl.when(cid == 0)
        def _():
            ring(feat_lo)

        @pl.when(cid == 1)
        def _():
            ring(feat_hi)

        plsc.subcore_barrier()
        if not fused_fin:
            def wout(out_hbm):
                for t in range(ROWS_T // 128):
                    base = sid * ROWS_T + t * 128
                    pltpu.sync_copy(acc_sh.at[pl.ds(base, 128)],
                                    out_hbm.at[pl.ds(base, 128)])

            @pl.when(cid == 0)
            def _():
                wout(out_lo)

            @pl.when(cid == 1)
            def _():
                wout(out_hi)
        else:
            pltpu.sync_copy(bias_hbm.at[cid], bias_v)

            @pl.when(sid < NS - 1)
            def _():
                pltpu.sync_copy(norm_hbm.at[pl.ds(sid * ROWS_T, ROWS_T)],
                                norm_v)

            @pl.when(sid == NS - 1)
            def _():
                pltpu.sync_copy(norm_hbm.at[pl.ds((NS - 1) * ROWS_T,
                                                  N - (NS - 1) * ROWS_T)],
                                norm_v.at[pl.ds(0, N - (NS - 1) * ROWS_T)])

            def wblock(t, nr):
                s0 = sid * ROWS_T + t * 128
                pltpu.sync_copy(acc_sh.at[pl.ds(s0, nr)],
                                wbuf.at[pl.ds(0, nr)])

                @pl.loop(0, nr // 16)
                def _(g):
                    nv = norm_v[pl.ds(t * 128 + g * 16, 16)]
                    for k in range(16):
                        r = g * 16 + k
                        s = nv[k]
                        for c in range(DH // 16):
                            sl = pl.ds(c * 16, 16)
                            wbuf[r, sl] = wbuf[r, sl] * s + bias_v[sl]

                pltpu.sync_copy(wbuf.at[pl.ds(0, nr)],
                                out_hbm.at[pl.ds(s0, nr),
                                           pl.ds(cid * DH, DH)])

            full_blocks = (N - (NS - 1) * ROWS_T) // 128
            tail_rows = N - (NS - 1) * ROWS_T - full_blocks * 128

            @pl.when(sid < NS - 1)
            def _():
                for t in range(ROWS_T // 128):
                    wblock(t, 128)

            @pl.when(sid == NS - 1)
            def _():
                for t in range(full_blocks):
                    wblock(t, 128)
                wblock(full_blocks, tail_rows)

    if fused_fin:
        out_type = jax.ShapeDtypeStruct((N, 2 * DH), jnp.float32)
        extra = [pltpu.VMEM((128, DH), jnp.float32),
                 pltpu.VMEM((ROWS_T,), jnp.float32),
                 pltpu.VMEM((DH,), jnp.float32)]
    else:
        out_type = [jax.ShapeDtypeStruct((NPAD, DH), jnp.float32),
                    jax.ShapeDtypeStruct((NPAD, DH), jnp.float32)]
        extra = []
    return functools.partial(
        pl.kernel,
        out_type=out_type,
        mesh=_mesh,
        scratch_types=[
            pltpu.VMEM((ETP,), jnp.int32),
            pltpu.VMEM((ETP,), jnp.int32),
            pltpu.VMEM((CHUNK, DH), jnp.float32),
            pltpu.VMEM((CHUNK, DH), jnp.float32),
            pltpu.VMEM((CHUNK, DH), jnp.float32),
            pltpu.VMEM((CHUNK, DH), jnp.float32),
        ] + extra + [
            pltpu.VMEM_SHARED((NPAD, DH), jnp.float32),
        ] + [pltpu.SemaphoreType.DMA] * 8,
        compiler_params=pltpu.CompilerParams(use_tc_tiling_on_sc=False),
    )(body)


_agg_hid = _make_agg(64)
_agg_out = _make_agg(32, fused_fin=True)


def _feat0_body(x_ref, deg_ref, degd1_ref, w_ref, lo_ref, hi_ref, n1_ref):
    norm = lax.rsqrt(jnp.maximum(deg_ref[...], 1.0))
    res = jnp.dot(x_ref[...] * norm, w_ref[...],
                  preferred_element_type=jnp.float32)
    lo_ref[...] = res[:, :64]
    hi_ref[...] = res[:, 64:]
    n1_ref[...] = lax.rsqrt(jnp.maximum(degd1_ref[...], 1.0))


def _mid_body(plo_ref, phi_ref, degi_ref, dego_ref, b0_ref, w1_ref,
              lo_ref, hi_ref):
    agg = jnp.concatenate([plo_ref[...], phi_ref[...]], axis=1)
    ni = lax.rsqrt(jnp.maximum(degi_ref[...], 1.0))
    h = jnp.maximum(agg * ni + b0_ref[...], 0.0)
    ns = lax.rsqrt(jnp.maximum(dego_ref[...], 1.0))
    res = jnp.dot(h * ns, w1_ref[...], preferred_element_type=jnp.float32)
    lo_ref[...] = res[:, :32]
    hi_ref[...] = res[:, 32:]


_BLK = 1000
_GRID = N // _BLK


def _feat0(x, deg, degd1, W0):
    return pl.pallas_call(
        _feat0_body,
        grid=(_GRID,),
        in_specs=[
            pl.BlockSpec((_BLK, 128), lambda i: (i, 0)),
            pl.BlockSpec((_BLK, 1), lambda i: (i, 0)),
            pl.BlockSpec((_BLK, 1), lambda i: (i, 0)),
            pl.BlockSpec((128, 128), lambda i: (0, 0)),
        ],
        out_specs=[
            pl.BlockSpec((_BLK, 64), lambda i: (i, 0)),
            pl.BlockSpec((_BLK, 64), lambda i: (i, 0)),
            pl.BlockSpec((_BLK, 1), lambda i: (i, 0)),
        ],
        out_shape=[
            jax.ShapeDtypeStruct((NF, 64), jnp.float32),
            jax.ShapeDtypeStruct((NF, 64), jnp.float32),
            jax.ShapeDtypeStruct((N, 1), jnp.float32),
        ],
    )(x, deg, degd1, W0)


def _mid(p_lo, p_hi, degi, dego, b0, W1):
    return pl.pallas_call(
        _mid_body,
        grid=(_GRID,),
        in_specs=[
            pl.BlockSpec((_BLK, 64), lambda i: (i, 0)),
            pl.BlockSpec((_BLK, 64), lambda i: (i, 0)),
            pl.BlockSpec((_BLK, 1), lambda i: (i, 0)),
            pl.BlockSpec((_BLK, 1), lambda i: (i, 0)),
            pl.BlockSpec((1, 128), lambda i: (0, 0)),
            pl.BlockSpec((128, 64), lambda i: (0, 0)),
        ],
        out_specs=[
            pl.BlockSpec((_BLK, 32), lambda i: (i, 0)),
            pl.BlockSpec((_BLK, 32), lambda i: (i, 0)),
        ],
        out_shape=[
            jax.ShapeDtypeStruct((NF, 32), jnp.float32),
            jax.ShapeDtypeStruct((NF, 32), jnp.float32),
        ],
    )(p_lo, p_hi, degi, dego, b0, W1)


def kernel(x, edge_index_l0, edge_index_l1, W0, b0, W1, b1):
    ei0 = edge_index_l0.astype(jnp.int32)
    ei1 = edge_index_l1.astype(jnp.int32)

    hist = _hist_call(ei0, ei1).reshape(4, NPAD)
    deg_src0 = hist[0].reshape(NPAD, 1)
    deg_dst0 = hist[1].reshape(NPAD, 1)
    deg_src1 = hist[2].reshape(NPAD, 1)
    deg_dst1 = hist[3].reshape(NPAD, 1)

    b1h = jnp.stack([b1[:32], b1[32:]])
    f0_lo, f0_hi, norm1 = _feat0(x, deg_src0, deg_dst1, W0)

    p0_lo, p0_hi = _agg_hid(f0_lo, f0_hi, ei0)
    f1_lo, f1_hi = _mid(p0_lo, p0_hi, deg_dst0, deg_src1,
                        b0.reshape(1, 128), W1)

    out = _agg_out(f1_lo, f1_hi, ei1, norm1.reshape(N), b1h)

    total_flops = float(
        (2 * 128 * E + 2 * 128 * 128 * N + N * 128) / 1e12
        + (2 * 128 * E + 2 * 128 * 64 * N + N * 128) / 1e12)
    return out, total_flops
